# baseline (speedup 1.0000x reference)
import functools

import jax
import jax.numpy as jnp
from jax.experimental import pallas as pl
from jax.experimental.pallas import tpu as pltpu

_B = 64
_C = 768
_L = 4
_HW2 = 14 * 14
_H = _C // 2
_BB = 8


def _gelu(x):
    return 0.5 * x * (1.0 + jax.lax.erf(x * (2.0 ** -0.5)))


def _router_body(p0, p1, p2, p3, w1, b1, w2, b2, w3, b3, out_w, out_c):
    inv = 1.0 / _HW2
    projs = []
    for p in (p0, p1, p2, p3):
        pooled = jnp.sum(p[...], axis=2) * inv
        z = jax.lax.dot_general(
            pooled, w1[...], (((1,), (1,)), ((), ())),
            preferred_element_type=jnp.float32) + b1[...]
        projs.append(_gelu(z))
    concat = jnp.concatenate(projs, axis=1)
    out_c[...] = concat
    hidden = _gelu(jax.lax.dot_general(
        concat, w2[...], (((1,), (1,)), ((), ())),
        preferred_element_type=jnp.float32) + b2[...])
    scores = jax.nn.sigmoid(jax.lax.dot_general(
        hidden, w3[...], (((1,), (1,)), ((), ())),
        preferred_element_type=jnp.float32) + b3[...])
    col = jax.lax.broadcasted_iota(jnp.int32, scores.shape, 1)
    s1 = jax.lax.slice(scores, (0, 1), (scores.shape[0], 2))
    s2 = jax.lax.slice(scores, (0, 2), (scores.shape[0], 3))
    keep1 = s1 >= s2
    mask = (col == 0) | (col == _L - 1) | ((col == 1) & keep1) | (
        (col == 2) & jnp.logical_not(keep1))
    w = scores * mask.astype(scores.dtype)
    out_w[...] = w / (jnp.sum(w, axis=1, keepdims=True) + 1e-6)


@functools.partial(jax.jit, static_argnums=())
def kernel(feat_0, prompt_0, prompt_1, prompt_2, prompt_3,
           W1, b1, W2, b2, W3, b3):
    del feat_0
    prompts = [p.reshape(_B, _C, _HW2)
               for p in (prompt_0, prompt_1, prompt_2, prompt_3)]
    grid = (_B // _BB,)
    p_spec = pl.BlockSpec((_BB, _C, _HW2), lambda i: (i, 0, 0))
    full = lambda *shape: pl.BlockSpec(shape, lambda i: (0,) * len(shape))
    out_w, out_c = pl.pallas_call(
        _router_body,
        grid=grid,
        in_specs=[
            p_spec, p_spec, p_spec, p_spec,
            full(_H, _C), full(1, _H),
            full(_C, _H * _L), full(1, _C),
            full(_L, _C), full(1, _L),
        ],
        out_specs=[
            pl.BlockSpec((_BB, _L), lambda i: (i, 0)),
            pl.BlockSpec((_BB, _H * _L), lambda i: (i, 0)),
        ],
        out_shape=[
            jax.ShapeDtypeStruct((_B, _L), jnp.float32),
            jax.ShapeDtypeStruct((_B, _H * _L), jnp.float32),
        ],
        compiler_params=pltpu.CompilerParams(
            dimension_semantics=("arbitrary",),
        ),
    )(*prompts, W1, b1.reshape(1, _H), W2, b2.reshape(1, _C),
      W3, b3.reshape(1, _L))
    return (out_w, out_c)
